# baseline (speedup 1.0000x reference)
import functools

import jax
import jax.numpy as jnp
from jax import lax
from jax.experimental import pallas as pl
from jax.experimental.pallas import tpu as pltpu
from jax.experimental.pallas import tpu_sc as plsc

NU = 25000
NI = 25000
PH = 25088
NN2 = 2 * PH
EMB = 64
E = 800000
EPAD = 802816
IR = EPAD // 128
IRT = IR // 16
RPT = 1568
SUB = 112
NSUB = RPT // SUB
LSUB = 49
NLSUB = RPT // LSUB
NLAYERS = 4

_MESH = plsc.VectorSubcoreMesh(core_axis_name="c", subcore_axis_name="s")
_CP = pltpu.CompilerParams(
    needs_layout_passes=False, use_tc_tiling_on_sc=False
)


def _rsqrt16(p):
    ib = plsc.bitcast(p, jnp.int32)
    seed = jnp.full((16,), 0x5F3759DF, jnp.int32) - lax.shift_right_arithmetic(
        ib, jnp.full((16,), 1, jnp.int32)
    )
    y = plsc.bitcast(seed, jnp.float32)
    for _ in range(3):
        y = y * (1.5 - 0.5 * p * y * y)
    return y


@functools.partial(
    pl.kernel,
    out_type=[
        jax.ShapeDtypeStruct((NN2, 16), jnp.float32),
        jax.ShapeDtypeStruct((NN2, EMB), jnp.float32),
    ],
    mesh=_MESH,
    compiler_params=_CP,
    scratch_types=[
        pltpu.VMEM_SHARED((PH, 16), jnp.float32),
        pltpu.VMEM((128, 16), jnp.float32),
        pltpu.VMEM((224, 16), jnp.float32),
        pltpu.VMEM((2, 128), jnp.int32),
        pltpu.VMEM((SUB, 16), jnp.float32),
        pltpu.VMEM((SUB, EMB), jnp.float32),
        pltpu.VMEM((SUB, EMB), jnp.float32),
        pltpu.VMEM((SUB, 16), jnp.float32),
    ],
)
def _init(ridx, ego, d_o, xs0_o, acc, ones, zb, gi, av, ev, xv, dv):
    c = lax.axis_index("c")
    s = lax.axis_index("s")

    def fill(i, _):
        ones[i, :] = jnp.full((16,), 1.0, jnp.float32)
        return 0

    lax.fori_loop(0, 128, fill, 0)

    def zfill(i, _):
        zb[i, :] = jnp.zeros((16,), jnp.float32)
        return 0

    lax.fori_loop(0, 224, zfill, 0)
    for q in range(RPT // 224):
        pltpu.sync_copy(zb, acc.at[pl.ds(s * RPT + q * 224, 224)])
    plsc.subcore_barrier()

    def edge(i, _):
        eb = c * IR + s * IRT + i
        pltpu.sync_copy(ridx.at[eb], gi)
        pltpu.sync_copy(ones, acc.at[gi.at[1]], add=True)
        return 0

    lax.fori_loop(0, IRT, edge, 0)
    plsc.subcore_barrier()

    def post(u, _):
        rb = s * RPT + u * SUB
        gb = c * PH + rb
        pltpu.sync_copy(acc.at[pl.ds(rb, SUB)], av)
        pltpu.sync_copy(ego.at[pl.ds(gb, SUB)], ev)

        def row(r, _):
            deg = av[r, :] + 1e-7
            d = _rsqrt16(deg)
            dv[r, :] = d
            for q in range(EMB // 16):
                xv[r, pl.ds(q * 16, 16)] = d * ev[r, pl.ds(q * 16, 16)]
            return 0

        lax.fori_loop(0, SUB, row, 0)
        pltpu.sync_copy(dv, d_o.at[pl.ds(gb, SUB)])
        pltpu.sync_copy(xv, xs0_o.at[pl.ds(gb, SUB)])
        return 0

    lax.fori_loop(0, NSUB, post, 0)


@functools.partial(
    pl.kernel,
    out_type=[
        jax.ShapeDtypeStruct((NN2, EMB), jnp.float32),
        jax.ShapeDtypeStruct((NN2, EMB), jnp.float32),
    ],
    mesh=_MESH,
    compiler_params=_CP,
    scratch_types=[
        pltpu.VMEM_SHARED((PH, EMB), jnp.float32),
        pltpu.VMEM((256, EMB), jnp.float32),
        pltpu.VMEM((2, 8, 2, 128), jnp.int32),
        pltpu.VMEM((LSUB, 16), jnp.float32),
        pltpu.SemaphoreType.DMA,
        pltpu.SemaphoreType.DMA,
        pltpu.SemaphoreType.DMA,
        pltpu.SemaphoreType.DMA,
        pltpu.SemaphoreType.DMA,
    ],
)
def _layer(xs, eidx, ego, d_n, sumi, xsn_o, sumo_o,
           acc, gbuf, bidx, dv, g0, g1, s0, s1, ix):
    c = lax.axis_index("c")
    s = lax.axis_index("s")

    bufa = gbuf.at[pl.ds(0, 128)]
    bufb = gbuf.at[pl.ds(128, 128)]

    def zfill(i, _):
        for q in range(EMB // 16):
            gbuf[i, pl.ds(q * 16, 16)] = jnp.zeros((16,), jnp.float32)
        return 0

    lax.fori_loop(0, 256, zfill, 0)
    for q in range(6):
        pltpu.sync_copy(gbuf, acc.at[pl.ds(s * RPT + q * 256, 256)])
    pltpu.sync_copy(gbuf.at[pl.ds(0, 32)], acc.at[pl.ds(s * RPT + 1536, 32)])
    plsc.subcore_barrier()

    eb0 = c * IR + s * IRT
    pltpu.sync_copy(eidx.at[pl.ds(eb0, 8), :, :], bidx.at[0])
    pltpu.async_copy(eidx.at[pl.ds(eb0 + 8, 8), :, :], bidx.at[1], ix)

    def superchunk(j, _):
        jp = lax.rem(j, 2)
        blk = bidx.at[jp]

        @pl.when(j > 0)
        def _():
            pltpu.make_async_copy(
                eidx.at[pl.ds(eb0 + j * 8, 8), :, :], bidx.at[jp], ix
            ).wait()

        @pl.when(j < IRT // 8 - 1)
        def _():
            pltpu.async_copy(
                eidx.at[pl.ds(eb0 + (j + 1) * 8, 8), :, :], bidx.at[1 - jp], ix
            )

        pltpu.async_copy(xs.at[blk.at[0, 0]], bufa, g0)
        pltpu.async_copy(xs.at[blk.at[1, 0]], bufb, g1)

        def pair(ii, _):
            ka = ii * 2
            pltpu.make_async_copy(xs.at[blk.at[ka, 0]], bufa, g0).wait()
            pltpu.async_copy(bufa, acc.at[blk.at[ka, 1]], s0, add=True)
            pltpu.make_async_copy(xs.at[blk.at[ka + 1, 0]], bufb, g1).wait()
            pltpu.async_copy(bufb, acc.at[blk.at[ka + 1, 1]], s1, add=True)
            pltpu.make_async_copy(bufa, acc.at[blk.at[ka, 1]], s0).wait()
            pltpu.async_copy(xs.at[blk.at[ka + 2, 0]], bufa, g0)
            pltpu.make_async_copy(bufb, acc.at[blk.at[ka + 1, 1]], s1).wait()
            pltpu.async_copy(xs.at[blk.at[ka + 3, 0]], bufb, g1)
            return 0

        lax.fori_loop(0, 3, pair, 0)
        pltpu.make_async_copy(xs.at[blk.at[6, 0]], bufa, g0).wait()
        pltpu.async_copy(bufa, acc.at[blk.at[6, 1]], s0, add=True)
        pltpu.make_async_copy(xs.at[blk.at[7, 0]], bufb, g1).wait()
        pltpu.async_copy(bufb, acc.at[blk.at[7, 1]], s1, add=True)
        pltpu.make_async_copy(bufa, acc.at[blk.at[6, 1]], s0).wait()
        pltpu.make_async_copy(bufb, acc.at[blk.at[7, 1]], s1).wait()
        return 0

    lax.fori_loop(0, IRT // 8, superchunk, 0)
    plsc.subcore_barrier()

    def post(u, _):
        rb = s * RPT + u * LSUB
        gb = c * PH + rb
        pltpu.sync_copy(acc.at[pl.ds(rb, LSUB)], gbuf.at[pl.ds(0, LSUB)])
        pltpu.sync_copy(ego.at[pl.ds(gb, LSUB)], gbuf.at[pl.ds(49, LSUB)])
        pltpu.sync_copy(sumi.at[pl.ds(gb, LSUB)], gbuf.at[pl.ds(98, LSUB)])
        pltpu.sync_copy(d_n.at[pl.ds(gb, LSUB)], dv)

        def row(r, _):
            d = dv[r, :]
            zs = []
            num = jnp.zeros((16,), jnp.float32)
            nz = jnp.zeros((16,), jnp.float32)
            ne = jnp.zeros((16,), jnp.float32)
            for q in range(EMB // 16):
                e = gbuf[49 + r, pl.ds(q * 16, 16)]
                z = d * gbuf[r, pl.ds(q * 16, 16)]
                zs.append(z)
                num = num + z * e
                nz = nz + z * z
                ne = ne + e * e
            num_s = jnp.sum(num)
            nz_s = jnp.sum(nz)
            ne_s = jnp.sum(ne)
            p = jnp.broadcast_to(jnp.maximum(nz_s * ne_s, 1e-30), (16,))
            rs = _rsqrt16(p)
            denom = jnp.maximum(p * rs, 1e-8)
            w = jnp.broadcast_to(num_s, (16,)) / denom
            w2 = w * d
            for q in range(EMB // 16):
                sc = gbuf[98 + r, pl.ds(q * 16, 16)]
                gbuf[147 + r, pl.ds(q * 16, 16)] = sc + w * zs[q]
                gbuf[196 + r, pl.ds(q * 16, 16)] = w2 * zs[q]
            return 0

        lax.fori_loop(0, LSUB, row, 0)
        pltpu.sync_copy(gbuf.at[pl.ds(147, LSUB)], sumo_o.at[pl.ds(gb, LSUB)])
        pltpu.sync_copy(gbuf.at[pl.ds(196, LSUB)], xsn_o.at[pl.ds(gb, LSUB)])
        return 0

    lax.fori_loop(0, NLSUB, post, 0)


@jax.jit
def kernel(user_emb, item_emb, rows, cols):
    ue = jnp.pad(user_emb, ((0, PH - NU), (0, 0)))
    ie = jnp.pad(item_emb, ((0, PH - NI), (0, 0)))
    ego = jnp.concatenate([ue, ie], axis=0)
    pad = jnp.full((EPAD - E,), PH - 1, jnp.int32)
    rp = jnp.concatenate([rows, pad])
    cp = jnp.concatenate([cols, pad])
    gat = jnp.concatenate([cp + PH, rp]).reshape(2 * IR, 1, 128)
    sct = jnp.concatenate([rp, cp]).reshape(2 * IR, 1, 128)
    eidx = jnp.concatenate([gat, sct], axis=1)
    d_n, xs = _init(eidx, ego)
    summ = jnp.zeros((NN2, EMB), jnp.float32)
    for _ in range(NLAYERS):
        xs, summ = _layer(xs, eidx, ego, d_n, summ)
    return summ[:NU], summ[PH:PH + NI]


# speedup vs baseline: 1.0014x; 1.0014x over previous
import functools

import jax
import jax.numpy as jnp
from jax import lax
from jax.experimental import pallas as pl
from jax.experimental.pallas import tpu as pltpu
from jax.experimental.pallas import tpu_sc as plsc

NU = 25000
NI = 25000
PH = 25088
NN2 = 2 * PH
EMB = 64
E = 800000
EPAD = 802816
IR = EPAD // 128
IRT = IR // 16
RPT = 1568
SUB = 112
NSUB = RPT // SUB
LSUB = 49
NLSUB = RPT // LSUB
NLAYERS = 4

_MESH = plsc.VectorSubcoreMesh(core_axis_name="c", subcore_axis_name="s")
_CP = pltpu.CompilerParams(
    needs_layout_passes=False, use_tc_tiling_on_sc=False
)


def _rsqrt16(p):
    ib = plsc.bitcast(p, jnp.int32)
    seed = jnp.full((16,), 0x5F3759DF, jnp.int32) - lax.shift_right_arithmetic(
        ib, jnp.full((16,), 1, jnp.int32)
    )
    y = plsc.bitcast(seed, jnp.float32)
    for _ in range(3):
        y = y * (1.5 - 0.5 * p * y * y)
    return y


@functools.partial(
    pl.kernel,
    out_type=[
        jax.ShapeDtypeStruct((NN2, 16), jnp.float32),
        jax.ShapeDtypeStruct((NN2, EMB), jnp.float32),
    ],
    mesh=_MESH,
    compiler_params=_CP,
    scratch_types=[
        pltpu.VMEM_SHARED((PH, 16), jnp.float32),
        pltpu.VMEM((128, 16), jnp.float32),
        pltpu.VMEM((224, 16), jnp.float32),
        pltpu.VMEM((2, 128), jnp.int32),
        pltpu.VMEM((SUB, 16), jnp.float32),
        pltpu.VMEM((SUB, EMB), jnp.float32),
        pltpu.VMEM((SUB, EMB), jnp.float32),
        pltpu.VMEM((SUB, 16), jnp.float32),
    ],
)
def _init(ridx, ego, d_o, xs0_o, acc, ones, zb, gi, av, ev, xv, dv):
    c = lax.axis_index("c")
    s = lax.axis_index("s")

    def fill(i, _):
        ones[i, :] = jnp.full((16,), 1.0, jnp.float32)
        return 0

    lax.fori_loop(0, 128, fill, 0)

    def zfill(i, _):
        zb[i, :] = jnp.zeros((16,), jnp.float32)
        return 0

    lax.fori_loop(0, 224, zfill, 0)
    for q in range(RPT // 224):
        pltpu.sync_copy(zb, acc.at[pl.ds(s * RPT + q * 224, 224)])
    plsc.subcore_barrier()

    def edge(i, _):
        eb = c * IR + s * IRT + i
        pltpu.sync_copy(ridx.at[eb], gi)
        pltpu.sync_copy(ones, acc.at[gi.at[1]], add=True)
        return 0

    lax.fori_loop(0, IRT, edge, 0)
    plsc.subcore_barrier()

    def post(u, _):
        rb = s * RPT + u * SUB
        gb = c * PH + rb
        pltpu.sync_copy(acc.at[pl.ds(rb, SUB)], av)
        pltpu.sync_copy(ego.at[pl.ds(gb, SUB)], ev)

        def row(r, _):
            deg = av[r, :] + 1e-7
            d = _rsqrt16(deg)
            dv[r, :] = d
            for q in range(EMB // 16):
                xv[r, pl.ds(q * 16, 16)] = d * ev[r, pl.ds(q * 16, 16)]
            return 0

        lax.fori_loop(0, SUB, row, 0)
        pltpu.sync_copy(dv, d_o.at[pl.ds(gb, SUB)])
        pltpu.sync_copy(xv, xs0_o.at[pl.ds(gb, SUB)])
        return 0

    lax.fori_loop(0, NSUB, post, 0)


@functools.partial(
    pl.kernel,
    out_type=[
        jax.ShapeDtypeStruct((NN2, EMB), jnp.float32),
        jax.ShapeDtypeStruct((NN2, EMB), jnp.float32),
    ],
    mesh=_MESH,
    compiler_params=_CP,
    scratch_types=[
        pltpu.VMEM_SHARED((PH, EMB), jnp.float32),
        pltpu.VMEM((256, EMB), jnp.float32),
        pltpu.VMEM((2, 8, 2, 128), jnp.int32),
        pltpu.VMEM((LSUB, 16), jnp.float32),
        pltpu.SemaphoreType.DMA,
        pltpu.SemaphoreType.DMA,
        pltpu.SemaphoreType.DMA,
        pltpu.SemaphoreType.DMA,
        pltpu.SemaphoreType.DMA,
    ],
)
def _layer(xs, eidx, ego, d_n, sumi, xsn_o, sumo_o,
           acc, gbuf, bidx, dv, g0, g1, s0, s1, ix):
    c = lax.axis_index("c")
    s = lax.axis_index("s")

    bufa = gbuf.at[pl.ds(0, 128)]
    bufb = gbuf.at[pl.ds(128, 128)]

    def zfill(i, _):
        for q in range(EMB // 16):
            gbuf[i, pl.ds(q * 16, 16)] = jnp.zeros((16,), jnp.float32)
        return 0

    lax.fori_loop(0, 256, zfill, 0)
    for q in range(6):
        pltpu.sync_copy(gbuf, acc.at[pl.ds(s * RPT + q * 256, 256)])
    pltpu.sync_copy(gbuf.at[pl.ds(0, 32)], acc.at[pl.ds(s * RPT + 1536, 32)])
    plsc.subcore_barrier()

    eb0 = c * IR + s * IRT
    pltpu.sync_copy(eidx.at[pl.ds(eb0, 8), :, :], bidx.at[0])

    def superchunk(j, _):
        jp = lax.rem(j, 2)
        blk = bidx.at[jp]

        @pl.when(j > 0)
        def _():
            pltpu.make_async_copy(
                eidx.at[pl.ds(eb0 + j * 8, 8), :, :], bidx.at[jp], ix
            ).wait()

        @pl.when(j < IRT // 8 - 1)
        def _():
            pltpu.async_copy(
                eidx.at[pl.ds(eb0 + (j + 1) * 8, 8), :, :], bidx.at[1 - jp], ix
            )

        pltpu.async_copy(xs.at[blk.at[0, 0]], bufa, g0)
        pltpu.async_copy(xs.at[blk.at[1, 0]], bufb, g1)

        def pair(ii, _):
            ka = ii * 2
            pltpu.make_async_copy(xs.at[blk.at[ka, 0]], bufa, g0).wait()
            pltpu.async_copy(bufa, acc.at[blk.at[ka, 1]], s0, add=True)
            pltpu.make_async_copy(xs.at[blk.at[ka + 1, 0]], bufb, g1).wait()
            pltpu.async_copy(bufb, acc.at[blk.at[ka + 1, 1]], s1, add=True)
            pltpu.make_async_copy(bufa, acc.at[blk.at[ka, 1]], s0).wait()
            pltpu.async_copy(xs.at[blk.at[ka + 2, 0]], bufa, g0)
            pltpu.make_async_copy(bufb, acc.at[blk.at[ka + 1, 1]], s1).wait()
            pltpu.async_copy(xs.at[blk.at[ka + 3, 0]], bufb, g1)
            return 0

        lax.fori_loop(0, 3, pair, 0)
        pltpu.make_async_copy(xs.at[blk.at[6, 0]], bufa, g0).wait()
        pltpu.async_copy(bufa, acc.at[blk.at[6, 1]], s0, add=True)
        pltpu.make_async_copy(xs.at[blk.at[7, 0]], bufb, g1).wait()
        pltpu.async_copy(bufb, acc.at[blk.at[7, 1]], s1, add=True)
        pltpu.make_async_copy(bufa, acc.at[blk.at[6, 1]], s0).wait()
        pltpu.make_async_copy(bufb, acc.at[blk.at[7, 1]], s1).wait()
        return 0

    lax.fori_loop(0, IRT // 8, superchunk, 0)
    plsc.subcore_barrier()

    def post(u, _):
        rb = s * RPT + u * LSUB
        gb = c * PH + rb
        pltpu.sync_copy(acc.at[pl.ds(rb, LSUB)], gbuf.at[pl.ds(0, LSUB)])
        pltpu.sync_copy(ego.at[pl.ds(gb, LSUB)], gbuf.at[pl.ds(49, LSUB)])
        pltpu.sync_copy(sumi.at[pl.ds(gb, LSUB)], gbuf.at[pl.ds(98, LSUB)])
        pltpu.sync_copy(d_n.at[pl.ds(gb, LSUB)], dv)

        def row(r, _):
            d = dv[r, :]
            zs = []
            num = jnp.zeros((16,), jnp.float32)
            nz = jnp.zeros((16,), jnp.float32)
            ne = jnp.zeros((16,), jnp.float32)
            for q in range(EMB // 16):
                e = gbuf[49 + r, pl.ds(q * 16, 16)]
                z = d * gbuf[r, pl.ds(q * 16, 16)]
                zs.append(z)
                num = num + z * e
                nz = nz + z * z
                ne = ne + e * e
            num_s = jnp.sum(num)
            nz_s = jnp.sum(nz)
            ne_s = jnp.sum(ne)
            p = jnp.broadcast_to(jnp.maximum(nz_s * ne_s, 1e-30), (16,))
            rs = _rsqrt16(p)
            denom = jnp.maximum(p * rs, 1e-8)
            w = jnp.broadcast_to(num_s, (16,)) / denom
            w2 = w * d
            for q in range(EMB // 16):
                sc = gbuf[98 + r, pl.ds(q * 16, 16)]
                gbuf[147 + r, pl.ds(q * 16, 16)] = sc + w * zs[q]
                gbuf[196 + r, pl.ds(q * 16, 16)] = w2 * zs[q]
            return 0

        lax.fori_loop(0, LSUB, row, 0)
        pltpu.sync_copy(gbuf.at[pl.ds(147, LSUB)], sumo_o.at[pl.ds(gb, LSUB)])
        pltpu.sync_copy(gbuf.at[pl.ds(196, LSUB)], xsn_o.at[pl.ds(gb, LSUB)])
        return 0

    lax.fori_loop(0, NLSUB, post, 0)


@jax.jit
def kernel(user_emb, item_emb, rows, cols):
    ue = jnp.pad(user_emb, ((0, PH - NU), (0, 0)))
    ie = jnp.pad(item_emb, ((0, PH - NI), (0, 0)))
    ego = jnp.concatenate([ue, ie], axis=0)
    pad = jnp.full((EPAD - E,), PH - 1, jnp.int32)
    rp = jnp.concatenate([rows, pad])
    cp = jnp.concatenate([cols, pad])
    gat = jnp.concatenate([cp + PH, rp]).reshape(2 * IR, 1, 128)
    sct = jnp.concatenate([rp, cp]).reshape(2 * IR, 1, 128)
    eidx = jnp.concatenate([gat, sct], axis=1)
    d_n, xs = _init(eidx, ego)
    summ = jnp.zeros((NN2, EMB), jnp.float32)
    for _ in range(NLAYERS):
        xs, summ = _layer(xs, eidx, ego, d_n, summ)
    return summ[:NU], summ[PH:PH + NI]


# speedup vs baseline: 1.0122x; 1.0108x over previous
import functools

import jax
import jax.numpy as jnp
from jax import lax
from jax.experimental import pallas as pl
from jax.experimental.pallas import tpu as pltpu
from jax.experimental.pallas import tpu_sc as plsc

NU = 25000
NI = 25000
PH = 25088
NN2 = 2 * PH
EMB = 64
E = 800000
EPAD = 802816
IR = EPAD // 128
IRT = IR // 16
RPT = 1568
SUB = 112
NSUB = RPT // SUB
LSUB = 49
NLSUB = RPT // LSUB
NLAYERS = 4

_MESH = plsc.VectorSubcoreMesh(core_axis_name="c", subcore_axis_name="s")
_CP = pltpu.CompilerParams(
    needs_layout_passes=False, use_tc_tiling_on_sc=False
)


def _rsqrt16(p):
    ib = plsc.bitcast(p, jnp.int32)
    seed = jnp.full((16,), 0x5F3759DF, jnp.int32) - lax.shift_right_arithmetic(
        ib, jnp.full((16,), 1, jnp.int32)
    )
    y = plsc.bitcast(seed, jnp.float32)
    for _ in range(3):
        y = y * (1.5 - 0.5 * p * y * y)
    return y


@functools.partial(
    pl.kernel,
    out_type=[
        jax.ShapeDtypeStruct((NN2, 16), jnp.float32),
        jax.ShapeDtypeStruct((NN2, EMB), jnp.float32),
    ],
    mesh=_MESH,
    compiler_params=_CP,
    scratch_types=[
        pltpu.VMEM_SHARED((PH, 16), jnp.float32),
        pltpu.VMEM((128, 16), jnp.float32),
        pltpu.VMEM((224, 16), jnp.float32),
        pltpu.VMEM((2, 128), jnp.int32),
        pltpu.VMEM((SUB, 16), jnp.float32),
        pltpu.VMEM((SUB, EMB), jnp.float32),
        pltpu.VMEM((SUB, EMB), jnp.float32),
        pltpu.VMEM((SUB, 16), jnp.float32),
    ],
)
def _init(ridx, ego, d_o, xs0_o, acc, ones, zb, gi, av, ev, xv, dv):
    c = lax.axis_index("c")
    s = lax.axis_index("s")

    def fill(i, _):
        ones[i, :] = jnp.full((16,), 1.0, jnp.float32)
        return 0

    lax.fori_loop(0, 128, fill, 0)

    def zfill(i, _):
        zb[i, :] = jnp.zeros((16,), jnp.float32)
        return 0

    lax.fori_loop(0, 224, zfill, 0)
    for q in range(RPT // 224):
        pltpu.sync_copy(zb, acc.at[pl.ds(s * RPT + q * 224, 224)])
    plsc.subcore_barrier()

    def edge(i, _):
        eb = c * IR + s * IRT + i
        pltpu.sync_copy(ridx.at[eb], gi)
        pltpu.sync_copy(ones, acc.at[gi.at[1]], add=True)
        return 0

    lax.fori_loop(0, IRT, edge, 0)
    plsc.subcore_barrier()

    def post(u, _):
        rb = s * RPT + u * SUB
        gb = c * PH + rb
        pltpu.sync_copy(acc.at[pl.ds(rb, SUB)], av)
        pltpu.sync_copy(ego.at[pl.ds(gb, SUB)], ev)

        def row(r, _):
            deg = av[r, :] + 1e-7
            d = _rsqrt16(deg)
            dv[r, :] = d
            for q in range(EMB // 16):
                xv[r, pl.ds(q * 16, 16)] = d * ev[r, pl.ds(q * 16, 16)]
            return 0

        lax.fori_loop(0, SUB, row, 0)
        pltpu.sync_copy(dv, d_o.at[pl.ds(gb, SUB)])
        pltpu.sync_copy(xv, xs0_o.at[pl.ds(gb, SUB)])
        return 0

    lax.fori_loop(0, NSUB, post, 0)


@functools.partial(
    pl.kernel,
    out_type=[
        jax.ShapeDtypeStruct((NN2, EMB), jnp.float32),
        jax.ShapeDtypeStruct((NN2, EMB), jnp.float32),
    ],
    mesh=_MESH,
    compiler_params=_CP,
    scratch_types=[
        pltpu.VMEM_SHARED((PH, EMB), jnp.float32),
        pltpu.VMEM((256, EMB), jnp.float32),
        pltpu.VMEM((2, 8, 2, 128), jnp.int32),
        pltpu.VMEM((LSUB, 16), jnp.float32),
        pltpu.SemaphoreType.DMA,
        pltpu.SemaphoreType.DMA,
        pltpu.SemaphoreType.DMA,
        pltpu.SemaphoreType.DMA,
        pltpu.SemaphoreType.DMA,
    ],
)
def _layer(xs, eidx, ego, d_n, sumi, xsn_o, sumo_o,
           acc, gbuf, bidx, dv, g0, g1, s0, s1, ix):
    c = lax.axis_index("c")
    s = lax.axis_index("s")

    bufa = gbuf.at[pl.ds(0, 128)]
    bufb = gbuf.at[pl.ds(128, 128)]

    def zfill(i, _):
        for q in range(EMB // 16):
            gbuf[i, pl.ds(q * 16, 16)] = jnp.zeros((16,), jnp.float32)
        return 0

    lax.fori_loop(0, 256, zfill, 0)
    for q in range(6):
        pltpu.sync_copy(gbuf, acc.at[pl.ds(s * RPT + q * 256, 256)])
    pltpu.sync_copy(gbuf.at[pl.ds(0, 32)], acc.at[pl.ds(s * RPT + 1536, 32)])
    plsc.subcore_barrier()

    eb0 = c * IR + s * IRT

    def _row(k, which):
        return bidx.at[lax.rem(k // 8, 2), lax.rem(k, 8), which]

    pltpu.sync_copy(eidx.at[pl.ds(eb0, 8), :, :], bidx.at[0])
    pltpu.async_copy(eidx.at[pl.ds(eb0 + 8, 8), :, :], bidx.at[1], ix)
    pltpu.async_copy(xs.at[_row(0, 0)], bufa, g0)
    pltpu.async_copy(xs.at[_row(1, 0)], bufb, g1)

    def pair(i, _):
        ka = i * 2
        pltpu.make_async_copy(xs.at[_row(ka, 0)], bufa, g0).wait()
        pltpu.async_copy(bufa, acc.at[_row(ka, 1)], s0, add=True)
        pltpu.make_async_copy(xs.at[_row(ka + 1, 0)], bufb, g1).wait()
        pltpu.async_copy(bufb, acc.at[_row(ka + 1, 1)], s1, add=True)

        @pl.when(lax.rem(i, 4) == 3)
        def _():
            j1 = i // 4 + 1
            pltpu.make_async_copy(
                eidx.at[pl.ds(eb0 + j1 * 8, 8), :, :],
                bidx.at[lax.rem(j1, 2)], ix,
            ).wait()

        pltpu.make_async_copy(bufa, acc.at[_row(ka, 1)], s0).wait()
        pltpu.async_copy(xs.at[_row(ka + 2, 0)], bufa, g0)
        pltpu.make_async_copy(bufb, acc.at[_row(ka + 1, 1)], s1).wait()
        pltpu.async_copy(xs.at[_row(ka + 3, 0)], bufb, g1)

        @pl.when(lax.rem(i, 4) == 3)
        def _():
            j2 = i // 4 + 2

            @pl.when(j2 <= IRT // 8 - 1)
            def _():
                pltpu.async_copy(
                    eidx.at[pl.ds(eb0 + j2 * 8, 8), :, :],
                    bidx.at[lax.rem(j2, 2)], ix,
                )

        return 0

    lax.fori_loop(0, IRT // 2 - 1, pair, 0)
    pltpu.make_async_copy(xs.at[_row(IRT - 2, 0)], bufa, g0).wait()
    pltpu.async_copy(bufa, acc.at[_row(IRT - 2, 1)], s0, add=True)
    pltpu.make_async_copy(xs.at[_row(IRT - 1, 0)], bufb, g1).wait()
    pltpu.async_copy(bufb, acc.at[_row(IRT - 1, 1)], s1, add=True)
    pltpu.make_async_copy(bufa, acc.at[_row(IRT - 2, 1)], s0).wait()
    pltpu.make_async_copy(bufb, acc.at[_row(IRT - 1, 1)], s1).wait()
    plsc.subcore_barrier()

    def post(u, _):
        rb = s * RPT + u * LSUB
        gb = c * PH + rb
        pltpu.sync_copy(acc.at[pl.ds(rb, LSUB)], gbuf.at[pl.ds(0, LSUB)])
        pltpu.sync_copy(ego.at[pl.ds(gb, LSUB)], gbuf.at[pl.ds(49, LSUB)])
        pltpu.sync_copy(sumi.at[pl.ds(gb, LSUB)], gbuf.at[pl.ds(98, LSUB)])
        pltpu.sync_copy(d_n.at[pl.ds(gb, LSUB)], dv)

        def row(r, _):
            d = dv[r, :]
            zs = []
            num = jnp.zeros((16,), jnp.float32)
            nz = jnp.zeros((16,), jnp.float32)
            ne = jnp.zeros((16,), jnp.float32)
            for q in range(EMB // 16):
                e = gbuf[49 + r, pl.ds(q * 16, 16)]
                z = d * gbuf[r, pl.ds(q * 16, 16)]
                zs.append(z)
                num = num + z * e
                nz = nz + z * z
                ne = ne + e * e
            num_s = jnp.sum(num)
            nz_s = jnp.sum(nz)
            ne_s = jnp.sum(ne)
            p = jnp.broadcast_to(jnp.maximum(nz_s * ne_s, 1e-30), (16,))
            rs = _rsqrt16(p)
            denom = jnp.maximum(p * rs, 1e-8)
            w = jnp.broadcast_to(num_s, (16,)) / denom
            w2 = w * d
            for q in range(EMB // 16):
                sc = gbuf[98 + r, pl.ds(q * 16, 16)]
                gbuf[147 + r, pl.ds(q * 16, 16)] = sc + w * zs[q]
                gbuf[196 + r, pl.ds(q * 16, 16)] = w2 * zs[q]
            return 0

        lax.fori_loop(0, LSUB, row, 0)
        pltpu.sync_copy(gbuf.at[pl.ds(147, LSUB)], sumo_o.at[pl.ds(gb, LSUB)])
        pltpu.sync_copy(gbuf.at[pl.ds(196, LSUB)], xsn_o.at[pl.ds(gb, LSUB)])
        return 0

    lax.fori_loop(0, NLSUB, post, 0)


@jax.jit
def kernel(user_emb, item_emb, rows, cols):
    ue = jnp.pad(user_emb, ((0, PH - NU), (0, 0)))
    ie = jnp.pad(item_emb, ((0, PH - NI), (0, 0)))
    ego = jnp.concatenate([ue, ie], axis=0)
    pad = jnp.full((EPAD - E,), PH - 1, jnp.int32)
    rp = jnp.concatenate([rows, pad])
    cp = jnp.concatenate([cols, pad])
    gat = jnp.concatenate([cp + PH, rp]).reshape(2 * IR, 1, 128)
    sct = jnp.concatenate([rp, cp]).reshape(2 * IR, 1, 128)
    eidx = jnp.concatenate([gat, sct], axis=1)
    d_n, xs = _init(eidx, ego)
    summ = jnp.zeros((NN2, EMB), jnp.float32)
    for _ in range(NLAYERS):
        xs, summ = _layer(xs, eidx, ego, d_n, summ)
    return summ[:NU], summ[PH:PH + NI]
